# Initial kernel scaffold; baseline (speedup 1.0000x reference)
import jax
import jax.numpy as jnp
from jax.experimental import pallas as pl


def kernel(points, centroids):
    raise NotImplementedError("write your pallas kernel here")


# speedup vs baseline: 13.7596x; 13.7596x over previous
import jax
import jax.numpy as jnp
from jax.experimental import pallas as pl

_BIG = 3.0e38
_B = 1024
_K = 1024
_D = 256
_BB = 256


def _kmeans_assign_kernel(x_ref, c_ref, o_ref):
    x = x_ref[:]
    c = c_ref[:]

    dots = jax.lax.dot_general(
        x, c, (((1,), (1,)), ((), ())),
        preferred_element_type=jnp.float32,
        precision=jax.lax.Precision.HIGHEST,
    )
    ones = jnp.ones((1, _D), dtype=jnp.float32)
    c_norm = jax.lax.dot_general(
        ones, c * c, (((1,), (1,)), ((), ())),
        preferred_element_type=jnp.float32,
        precision=jax.lax.Precision.HIGHEST,
    )
    scores = c_norm - 2.0 * dots

    col = jax.lax.broadcasted_iota(jnp.int32, (_BB, _K), 1)

    m1 = jnp.min(scores, axis=1, keepdims=True)
    i1 = jnp.min(jnp.where(scores == m1, col, _K), axis=1, keepdims=True)

    masked = jnp.where(col == i1, _BIG, scores)
    m2 = jnp.min(masked, axis=1, keepdims=True)
    i2 = jnp.min(jnp.where(masked == m2, col, _K), axis=1, keepdims=True)

    oh1 = (col == i1).astype(jnp.float32)
    oh2 = (col == i2).astype(jnp.float32)
    c1 = jax.lax.dot_general(oh1, c, (((1,), (0,)), ((), ())),
                             preferred_element_type=jnp.float32,
                             precision=jax.lax.Precision.HIGHEST)
    c2 = jax.lax.dot_general(oh2, c, (((1,), (0,)), ((), ())),
                             preferred_element_type=jnp.float32,
                             precision=jax.lax.Precision.HIGHEST)
    d1 = jnp.sum(jnp.square(x - c1), axis=1, keepdims=True)
    d2 = jnp.sum(jnp.square(x - c2), axis=1, keepdims=True)

    o_ref[:] = jnp.where(
        d1 < d2, i1, jnp.where(d2 < d1, i2, jnp.minimum(i1, i2))
    ).astype(jnp.int32)


def kernel(points, centroids):
    out = pl.pallas_call(
        _kmeans_assign_kernel,
        grid=(_B // _BB,),
        in_specs=[
            pl.BlockSpec((_BB, _D), lambda i: (i, 0)),
            pl.BlockSpec((_K, _D), lambda i: (0, 0)),
        ],
        out_specs=pl.BlockSpec((_BB, 1), lambda i: (i, 0)),
        out_shape=jax.ShapeDtypeStruct((_B, 1), jnp.int32),
    )(points, centroids)
    return out[:, 0]


# speedup vs baseline: 21.1348x; 1.5360x over previous
import jax
import jax.numpy as jnp
from jax.experimental import pallas as pl

_BIG = 3.0e38
_B = 1024
_K = 1024
_D = 256
_BB = 256


def _kmeans_assign_kernel(x_ref, c_ref, o_ref):
    x = x_ref[:]
    c = c_ref[:]

    def _split2(v):
        hi = v.astype(jnp.bfloat16)
        lo = (v - hi.astype(jnp.float32)).astype(jnp.bfloat16)
        return hi, lo

    x_hi, x_lo = _split2(x)
    c_hi, c_lo = _split2(c)

    def _mm_t(a, b):
        return jax.lax.dot_general(a, b, (((1,), (1,)), ((), ())),
                                   preferred_element_type=jnp.float32)

    dots = _mm_t(x_hi, c_hi) + (_mm_t(x_hi, c_lo) + _mm_t(x_lo, c_hi))

    cc = c * c
    cc_hi, cc_lo = _split2(cc)
    ones = jnp.ones((1, _D), dtype=jnp.bfloat16)
    c_norm = _mm_t(ones, cc_hi) + _mm_t(ones, cc_lo)
    scores = c_norm - 2.0 * dots

    col = jax.lax.broadcasted_iota(jnp.int32, (_BB, _K), 1)

    m1 = jnp.min(scores, axis=1, keepdims=True)
    i1 = jnp.min(jnp.where(scores == m1, col, _K), axis=1, keepdims=True)

    masked = jnp.where(col == i1, _BIG, scores)
    m2 = jnp.min(masked, axis=1, keepdims=True)
    i2 = jnp.min(jnp.where(masked == m2, col, _K), axis=1, keepdims=True)

    oh1 = (col == i1).astype(jnp.bfloat16)
    oh2 = (col == i2).astype(jnp.bfloat16)
    c_lo2 = (c - c_hi.astype(jnp.float32) - c_lo.astype(jnp.float32)
             ).astype(jnp.bfloat16)

    def _mm(a, b):
        return jax.lax.dot_general(a, b, (((1,), (0,)), ((), ())),
                                   preferred_element_type=jnp.float32)

    c1 = _mm(oh1, c_hi) + _mm(oh1, c_lo) + _mm(oh1, c_lo2)
    c2 = _mm(oh2, c_hi) + _mm(oh2, c_lo) + _mm(oh2, c_lo2)
    d1 = jnp.sum(jnp.square(x - c1), axis=1, keepdims=True)
    d2 = jnp.sum(jnp.square(x - c2), axis=1, keepdims=True)

    o_ref[:] = jnp.where(
        d1 < d2, i1, jnp.where(d2 < d1, i2, jnp.minimum(i1, i2))
    ).astype(jnp.int32)


def kernel(points, centroids):
    out = pl.pallas_call(
        _kmeans_assign_kernel,
        grid=(_B // _BB,),
        in_specs=[
            pl.BlockSpec((_BB, _D), lambda i: (i, 0)),
            pl.BlockSpec((_K, _D), lambda i: (0, 0)),
        ],
        out_specs=pl.BlockSpec((_BB, 1), lambda i: (i, 0)),
        out_shape=jax.ShapeDtypeStruct((_B, 1), jnp.int32),
    )(points, centroids)
    return out[:, 0]


# speedup vs baseline: 24.9074x; 1.1785x over previous
import jax
import jax.numpy as jnp
from jax.experimental import pallas as pl

_BIG = 3.0e38
_B = 1024
_K = 1024
_D = 256
_BB = 1024


def _kmeans_assign_kernel(x_ref, c_ref, o_ref):
    x = x_ref[:]
    c = c_ref[:]

    def _split2(v):
        hi = v.astype(jnp.bfloat16)
        lo = (v - hi.astype(jnp.float32)).astype(jnp.bfloat16)
        return hi, lo

    x_hi, x_lo = _split2(x)
    c_hi, c_lo = _split2(c)

    def _mm_t(a, b):
        return jax.lax.dot_general(a, b, (((1,), (1,)), ((), ())),
                                   preferred_element_type=jnp.float32)

    dots = _mm_t(x_hi, c_hi) + (_mm_t(x_hi, c_lo) + _mm_t(x_lo, c_hi))

    cc = c * c
    cc_hi, cc_lo = _split2(cc)
    ones = jnp.ones((1, _D), dtype=jnp.bfloat16)
    c_norm = _mm_t(ones, cc_hi) + _mm_t(ones, cc_lo)
    scores = c_norm - 2.0 * dots

    col = jax.lax.broadcasted_iota(jnp.int32, (_BB, _K), 1)

    m1 = jnp.min(scores, axis=1, keepdims=True)
    i1 = jnp.min(jnp.where(scores == m1, col, _K), axis=1, keepdims=True)

    masked = jnp.where(col == i1, _BIG, scores)
    m2 = jnp.min(masked, axis=1, keepdims=True)
    i2 = jnp.min(jnp.where(masked == m2, col, _K), axis=1, keepdims=True)

    oh1 = (col == i1).astype(jnp.bfloat16)
    oh2 = (col == i2).astype(jnp.bfloat16)
    c_lo2 = (c - c_hi.astype(jnp.float32) - c_lo.astype(jnp.float32)
             ).astype(jnp.bfloat16)

    def _mm(a, b):
        return jax.lax.dot_general(a, b, (((1,), (0,)), ((), ())),
                                   preferred_element_type=jnp.float32)

    c1 = _mm(oh1, c_hi) + _mm(oh1, c_lo) + _mm(oh1, c_lo2)
    c2 = _mm(oh2, c_hi) + _mm(oh2, c_lo) + _mm(oh2, c_lo2)
    d1 = jnp.sum(jnp.square(x - c1), axis=1, keepdims=True)
    d2 = jnp.sum(jnp.square(x - c2), axis=1, keepdims=True)

    o_ref[:] = jnp.where(
        d1 < d2, i1, jnp.where(d2 < d1, i2, jnp.minimum(i1, i2))
    ).astype(jnp.int32)


def kernel(points, centroids):
    out = pl.pallas_call(
        _kmeans_assign_kernel,
        grid=(_B // _BB,),
        in_specs=[
            pl.BlockSpec((_BB, _D), lambda i: (i, 0)),
            pl.BlockSpec((_K, _D), lambda i: (0, 0)),
        ],
        out_specs=pl.BlockSpec((_BB, 1), lambda i: (i, 0)),
        out_shape=jax.ShapeDtypeStruct((_B, 1), jnp.int32),
    )(points, centroids)
    return out[:, 0]
